# Initial kernel scaffold; baseline (speedup 1.0000x reference)
import jax
import jax.numpy as jnp
from jax.experimental import pallas as pl


def kernel(x, w1, b1, w2, b2, g1, beta1, g2, beta2, prelu_alpha):
    raise NotImplementedError("write your pallas kernel here")


# speedup vs baseline: 1.3090x; 1.3090x over previous
import functools
import itertools

import jax
import jax.numpy as jnp
from jax.experimental import pallas as pl
from jax.experimental.pallas import tpu as pltpu


def _round_up(x, m):
    return ((x + m - 1) // m) * m


def _conv1_kernel(cols_ref, w_ref, b_ref, o_ref, sum_ref, sq_ref):
    @pl.when(pl.program_id(1) == 0)
    def _():
        sum_ref[...] = jnp.zeros_like(sum_ref)
        sq_ref[...] = jnp.zeros_like(sq_ref)
    y = jnp.dot(cols_ref[...], w_ref[...],
                preferred_element_type=jnp.float32) + b_ref[...]
    o_ref[...] = y
    sum_ref[...] += jnp.sum(y, axis=0, keepdims=True)[None]
    sq_ref[...] += jnp.sum(y * y, axis=0, keepdims=True)[None]


def _conv2_kernel(lo_ref, mid_ref, hi_ref, sc_ref, sh_ref, al_ref,
                  w_ref, b_ref, o_ref, sum_ref, sq_ref, *, td, ho, wo):
    g = pl.program_id(1)
    ng = pl.num_programs(1)

    @pl.when(g == 0)
    def _():
        sum_ref[...] = jnp.zeros_like(sum_ref)
        sq_ref[...] = jnp.zeros_like(sq_ref)

    alpha = al_ref[0, 0]
    sc = sc_ref[...].reshape(1, 1, 1, 1, -1)
    sh = sh_ref[...].reshape(1, 1, 1, 1, -1)

    def norm(t):
        y = t * sc + sh
        return jnp.where(y > 0, y, alpha * y)

    z_lo = norm(lo_ref[...]) * jnp.where(g == 0, 0.0, 1.0)
    z_mid = norm(mid_ref[...])
    z_hi = norm(hi_ref[...]) * jnp.where(g == ng - 1, 0.0, 1.0)
    z = jnp.concatenate([z_lo, z_mid, z_hi], axis=1)[0]
    zp = jnp.pad(z, ((0, 0), (1, 1), (1, 1), (0, 0)))

    m = td * ho * wo
    cols = jnp.concatenate(
        [zp[a:a + td, b:b + ho, c:c + wo, :].reshape(m, -1)
         for a, b, c in itertools.product(range(3), range(3), range(3))],
        axis=-1)
    y = jnp.dot(cols, w_ref[...],
                preferred_element_type=jnp.float32) + b_ref[...]
    o_ref[...] = y[None]
    sum_ref[...] += jnp.sum(y, axis=0, keepdims=True)[None]
    sq_ref[...] += jnp.sum(y * y, axis=0, keepdims=True)[None]


def _final_kernel(y_ref, sc_ref, sh_ref, al_ref, o_ref):
    y = y_ref[0]
    t = y * sc_ref[0] + sh_ref[0]
    z = jnp.where(t > 0, t, al_ref[0, 0] * t)
    o_ref[...] = z.T[None]


def _fold_scale_shift(ssum, ssq, gamma, beta, s, eps=1e-5):
    mean = ssum[:, 0, :] / s
    var = jnp.maximum(ssq[:, 0, :] / s - mean * mean, 0.0)
    rstd = jax.lax.rsqrt(var + eps)
    scale = gamma[None, :] * rstd
    shift = beta[None, :] - mean * scale
    return scale.astype(jnp.float32), shift.astype(jnp.float32)


def kernel(x, w1, b1, w2, b2, g1, beta1, g2, beta2, prelu_alpha):
    N, Cin, D, H, W = x.shape
    k = 3
    C = w1.shape[-1]
    Do, Ho, Wo = D // 2, H // 2, W // 2
    S = Do * Ho * Wo
    M = N * S

    xt = jnp.transpose(x, (0, 2, 3, 4, 1))
    xp = jnp.pad(xt, ((0, 0), (1, 1), (1, 1), (1, 1), (0, 0)))
    taps = []
    for a in range(k):
        for b in range(k):
            for c in range(k):
                sl = jax.lax.slice(
                    xp, (0, a, b, c, 0),
                    (N, a + 2 * (Do - 1) + 1, b + 2 * (Ho - 1) + 1,
                     c + 2 * (Wo - 1) + 1, Cin),
                    (1, 2, 2, 2, 1))
                taps.append(sl.reshape(M, Cin))
    cols1 = jnp.concatenate(taps, axis=1)
    w1f = w1.reshape(k * k * k * Cin, C)
    b1f = b1.reshape(1, C).astype(jnp.float32)

    ts1 = min(4096, S)
    n_s1 = S // ts1
    alpha = jnp.asarray(prelu_alpha, jnp.float32).reshape(1, 1)

    y1, ssum1, ssq1 = pl.pallas_call(
        _conv1_kernel,
        out_shape=(jax.ShapeDtypeStruct((M, C), jnp.float32),
                   jax.ShapeDtypeStruct((N, 1, C), jnp.float32),
                   jax.ShapeDtypeStruct((N, 1, C), jnp.float32)),
        grid=(N, n_s1),
        in_specs=[
            pl.BlockSpec((ts1, cols1.shape[1]), lambda n, s: (n * n_s1 + s, 0)),
            pl.BlockSpec((w1f.shape[0], C), lambda n, s: (0, 0)),
            pl.BlockSpec((1, C), lambda n, s: (0, 0)),
        ],
        out_specs=(pl.BlockSpec((ts1, C), lambda n, s: (n * n_s1 + s, 0)),
                   pl.BlockSpec((1, 1, C), lambda n, s: (n, 0, 0)),
                   pl.BlockSpec((1, 1, C), lambda n, s: (n, 0, 0))),
        compiler_params=pltpu.CompilerParams(
            dimension_semantics=("parallel", "arbitrary")),
    )(cols1, w1f, b1f)

    scale1, shift1 = _fold_scale_shift(ssum1, ssq1, g1, beta1, S)

    y1r = y1.reshape(N, Do, Ho, Wo, C)
    w2f = w2.reshape(k * k * k * C, C)
    b2f = b2.reshape(1, C).astype(jnp.float32)
    td = 4 if Do % 4 == 0 else 1
    ng = Do // td

    y2, ssum2, ssq2 = pl.pallas_call(
        functools.partial(_conv2_kernel, td=td, ho=Ho, wo=Wo),
        out_shape=(jax.ShapeDtypeStruct((N, S, C), jnp.float32),
                   jax.ShapeDtypeStruct((N, 1, C), jnp.float32),
                   jax.ShapeDtypeStruct((N, 1, C), jnp.float32)),
        grid=(N, ng),
        in_specs=[
            pl.BlockSpec((1, 1, Ho, Wo, C),
                         lambda n, g: (n, jnp.maximum(g * td - 1, 0), 0, 0, 0)),
            pl.BlockSpec((1, td, Ho, Wo, C), lambda n, g: (n, g, 0, 0, 0)),
            pl.BlockSpec((1, 1, Ho, Wo, C),
                         lambda n, g: (n, jnp.minimum(g * td + td, Do - 1),
                                       0, 0, 0)),
            pl.BlockSpec((1, 1, C), lambda n, g: (n, 0, 0)),
            pl.BlockSpec((1, 1, C), lambda n, g: (n, 0, 0)),
            pl.BlockSpec((1, 1), lambda n, g: (0, 0)),
            pl.BlockSpec((w2f.shape[0], C), lambda n, g: (0, 0)),
            pl.BlockSpec((1, C), lambda n, g: (0, 0)),
        ],
        out_specs=(pl.BlockSpec((1, td * Ho * Wo, C), lambda n, g: (n, g, 0)),
                   pl.BlockSpec((1, 1, C), lambda n, g: (n, 0, 0)),
                   pl.BlockSpec((1, 1, C), lambda n, g: (n, 0, 0))),
        compiler_params=pltpu.CompilerParams(
            dimension_semantics=("parallel", "arbitrary")),
    )(y1r.reshape(N, Do, Ho, Wo, C), y1r, y1r, scale1.reshape(N, 1, C),
      shift1.reshape(N, 1, C), alpha, w2f, b2f)

    scale2, shift2 = _fold_scale_shift(ssum2, ssq2, g2, beta2, S)

    ts2 = min(4096, S)
    n_s2 = S // ts2
    out = pl.pallas_call(
        _final_kernel,
        out_shape=jax.ShapeDtypeStruct((N, C, S), jnp.float32),
        grid=(N, n_s2),
        in_specs=[
            pl.BlockSpec((1, ts2, C), lambda n, s: (n, s, 0)),
            pl.BlockSpec((1, 1, C), lambda n, s: (n, 0, 0)),
            pl.BlockSpec((1, 1, C), lambda n, s: (n, 0, 0)),
            pl.BlockSpec((1, 1), lambda n, s: (0, 0)),
        ],
        out_specs=pl.BlockSpec((1, C, ts2), lambda n, s: (n, 0, s)),
        compiler_params=pltpu.CompilerParams(
            dimension_semantics=("parallel", "parallel")),
    )(y2, scale2.reshape(N, 1, C), shift2.reshape(N, 1, C), alpha)

    return out.reshape(N, C, Do, Ho, Wo)
